# baseline (speedup 1.0000x reference)
import functools

import jax
import jax.numpy as jnp
from jax import lax
from jax.experimental import pallas as pl
from jax.experimental.pallas import tpu as pltpu
from jax.experimental.pallas import tpu_sc as plsc

VOCAB = 1000000
EMBED = 64
B = 4096
L = 200
EPS = 1e-5

BL = B * L
SR = BL // 2
NC, NS = 2, 16
NW = NC * NS
SR_PER_W = SR // NW
CS = 320
NCHUNK = SR_PER_W // CS
GS = 64
NSTR = 2 * (CS // GS)
NBUF = 2

TCBLK = 12800
TCGRID = SR // TCBLK


def _sc_body(ids_hbm, tok_hbm, out_hbm, idx_v, rows_v, sem0, sem1):
    wid = lax.axis_index("s") * NC + lax.axis_index("c")
    sems = (sem0, sem1)

    def start_gather(c, b):
        q0 = wid * (2 * TCBLK // GS) + c * (CS // GS)
        for h in range(2):
            pltpu.sync_copy(
                ids_hbm.at[pl.ds(q0 + h * (TCBLK // GS), CS // GS)],
                idx_v.at[pl.ds(b * NSTR + h * (CS // GS), CS // GS)])
        for j in range(NSTR):
            h, gg = j // (CS // GS), j % (CS // GS)
            pltpu.async_copy(
                tok_hbm.at[idx_v.at[b * NSTR + j]],
                rows_v.at[pl.ds((b * 2 + h) * CS + gg * GS, GS)],
                sems[b])

    def drain(b):
        pltpu.make_async_copy(
            tok_hbm.at[pl.ds(0, 2 * CS)],
            rows_v.at[pl.ds(b * 2 * CS, 2 * CS)], sems[b]).wait()

    def writeback(c, b):
        out_base = wid * SR_PER_W + c * CS
        for h in range(2):
            pltpu.sync_copy(
                rows_v.at[pl.ds((b * 2 + h) * CS, CS)],
                out_hbm.at[pl.ds(out_base, CS), pl.ds(64 * h, 64)])

    start_gather(0, 0)

    def pair_body(i, carry):
        c2 = i * NBUF
        for b in range(NBUF):
            c = c2 + b

            @pl.when(c < NCHUNK - 1)
            def _():
                start_gather(c + 1, (b + 1) % NBUF)
            drain(b)
            writeback(c, b)
        return carry
    lax.fori_loop(0, NCHUNK // NBUF, pair_body, 0)


def _ln_body(y_ref, pos_ref, m_ref, g_ref, b_ref, o_ref):
    x = y_ref[...] + jnp.tile(pos_ref[...], (TCBLK // L, 1))
    m = m_ref[...]
    mu = jnp.dot(x, m, preferred_element_type=jnp.float32)
    var = jnp.dot(x * x, m, preferred_element_type=jnp.float32) - mu * mu
    o = ((x - mu) * lax.rsqrt(var + EPS) * g_ref[0:1, :]
         + b_ref[0:1, :])
    o2 = jnp.concatenate([o[:, :EMBED], o[:, EMBED:]], axis=0)
    o_ref[...] = o2.reshape(2 * TCBLK // L, L, EMBED)


@jax.jit
def _call(ids, token_table, pos2, mmat, g2, b2):
    mesh = plsc.VectorSubcoreMesh(core_axis_name="c", subcore_axis_name="s")
    gather = functools.partial(
        pl.kernel,
        mesh=mesh,
        out_type=jax.ShapeDtypeStruct((SR, 128), jnp.float32),
        compiler_params=pltpu.CompilerParams(
            needs_layout_passes=False, use_tc_tiling_on_sc=False),
        scratch_types=[
            pltpu.VMEM((NBUF * NSTR, GS), jnp.int32),
            pltpu.VMEM((NBUF * 2 * CS, EMBED), jnp.float32),
            pltpu.SemaphoreType.DMA,
            pltpu.SemaphoreType.DMA,
        ],
    )(_sc_body)
    y = gather(ids, token_table)

    out = pl.pallas_call(
        _ln_body,
        grid=(TCGRID,),
        in_specs=[
            pl.BlockSpec((TCBLK, 128), lambda i: (i, 0)),
            pl.BlockSpec((L, 128), lambda i: (0, 0)),
            pl.BlockSpec((128, 128), lambda i: (0, 0)),
            pl.BlockSpec((8, 128), lambda i: (0, 0)),
            pl.BlockSpec((8, 128), lambda i: (0, 0)),
        ],
        out_specs=pl.BlockSpec((2 * TCBLK // L, L, EMBED), lambda i: (i, 0, 0)),
        out_shape=jax.ShapeDtypeStruct((B, L, EMBED), jnp.float32),
    )(y, pos2, mmat, g2, b2)
    return out


def kernel(input_ids, token_table, pos_table, ln_gamma, ln_beta):
    ids2 = input_ids.reshape(BL // GS, GS).astype(jnp.int32)
    pos2 = jnp.concatenate([pos_table[:L]] * 2, axis=1)
    eye2 = jnp.eye(2, dtype=jnp.float32)
    mmat = jnp.kron(eye2, jnp.full((EMBED, EMBED), 1.0 / EMBED, jnp.float32))
    g2 = jnp.tile(jnp.concatenate([ln_gamma, ln_gamma])[None, :], (8, 1))
    b2 = jnp.tile(jnp.concatenate([ln_beta, ln_beta])[None, :], (8, 1))
    return _call(ids2, token_table, pos2, mmat, g2, b2)
